# baseline (speedup 1.0000x reference)
import functools

import jax
import jax.numpy as jnp
from jax import lax
from jax.experimental import pallas as pl
from jax.experimental.pallas import tpu as pltpu
from jax.experimental.pallas import tpu_sc as plsc

N = 10000
E = 320000
D = 128
H = 128
B = 64
OUT = 2
E_TOT = E + N

MBLK = 1000
MSTEPS = N // MBLK


def _front_body(x_ref, w_ref, att_ref, h_ref, a_ref):
    h = jnp.dot(x_ref[...], w_ref[...], preferred_element_type=jnp.float32)
    h_ref[...] = h
    a_ref[...] = jnp.dot(h, att_ref[...], preferred_element_type=jnp.float32)


def _tc_front(x, w, att):
    return pl.pallas_call(
        _front_body,
        grid=(MSTEPS,),
        in_specs=[
            pl.BlockSpec((MBLK, D), lambda i: (i, 0)),
            pl.BlockSpec((D, H), lambda i: (0, 0)),
            pl.BlockSpec((H, 2), lambda i: (0, 0)),
        ],
        out_specs=[
            pl.BlockSpec((MBLK, H), lambda i: (i, 0)),
            pl.BlockSpec((MBLK, 2), lambda i: (i, 0)),
        ],
        out_shape=[
            jax.ShapeDtypeStruct((N, H), jnp.float32),
            jax.ShapeDtypeStruct((N, 2), jnp.float32),
        ],
    )(x, w, att)


def _mid_body(p_ref, d_ref, b_ref, w_ref, att_ref, h_ref, a_ref):
    s = p_ref[0] + p_ref[1]
    den = jnp.maximum(d_ref[0, :, 0:1] + d_ref[1, :, 0:1], 1e-16)
    y = jnp.maximum(s / den + b_ref[...], 0.0)
    h = jnp.dot(y, w_ref[...], preferred_element_type=jnp.float32)
    h_ref[...] = h
    a_ref[...] = jnp.dot(h, att_ref[...], preferred_element_type=jnp.float32)


def _tc_mid(parts, denoms, bias, w, att):
    return pl.pallas_call(
        _mid_body,
        grid=(MSTEPS,),
        in_specs=[
            pl.BlockSpec((2, MBLK, H), lambda i: (0, i, 0)),
            pl.BlockSpec((2, MBLK, 16), lambda i: (0, i, 0)),
            pl.BlockSpec((1, H), lambda i: (0, 0)),
            pl.BlockSpec((H, H), lambda i: (0, 0)),
            pl.BlockSpec((H, 2), lambda i: (0, 0)),
        ],
        out_specs=[
            pl.BlockSpec((MBLK, H), lambda i: (i, 0)),
            pl.BlockSpec((MBLK, 2), lambda i: (i, 0)),
        ],
        out_shape=[
            jax.ShapeDtypeStruct((N, H), jnp.float32),
            jax.ShapeDtypeStruct((N, 2), jnp.float32),
        ],
    )(parts, denoms, bias, w, att)


def _final_body(p_ref, d_ref, b_ref, batch_ref, wc_ref, bc_ref, o_ref,
                acc_ref, cnt_ref):
    i = pl.program_id(0)

    @pl.when(i == 0)
    def _():
        acc_ref[...] = jnp.zeros_like(acc_ref)
        cnt_ref[...] = jnp.zeros_like(cnt_ref)

    s = p_ref[0] + p_ref[1]
    den = jnp.maximum(d_ref[0, :, 0:1] + d_ref[1, :, 0:1], 1e-16)
    y = jnp.maximum(s / den + b_ref[...], 0.0)
    seg = batch_ref[0, 0, :]
    oh = (seg[None, :] == lax.broadcasted_iota(jnp.int32, (B, MBLK), 0)
          ).astype(jnp.float32)
    acc_ref[...] += jnp.dot(oh, y, preferred_element_type=jnp.float32)
    cnt_ref[...] += jnp.sum(oh, axis=1, keepdims=True)

    @pl.when(i == MSTEPS - 1)
    def _():
        pooled = acc_ref[...] / jnp.maximum(cnt_ref[...], 1.0)
        o_ref[...] = (jnp.dot(pooled, wc_ref[...],
                              preferred_element_type=jnp.float32)
                      + bc_ref[...])


def _tc_final(parts, denoms, bias, batch3, wc, bc):
    return pl.pallas_call(
        _final_body,
        grid=(MSTEPS,),
        in_specs=[
            pl.BlockSpec((2, MBLK, H), lambda i: (0, i, 0)),
            pl.BlockSpec((2, MBLK, 16), lambda i: (0, i, 0)),
            pl.BlockSpec((1, H), lambda i: (0, 0)),
            pl.BlockSpec((1, 1, MBLK), lambda i: (i, 0, 0)),
            pl.BlockSpec((H, OUT), lambda i: (0, 0)),
            pl.BlockSpec((1, OUT), lambda i: (0, 0)),
        ],
        out_specs=pl.BlockSpec((B, OUT), lambda i: (0, 0)),
        out_shape=jax.ShapeDtypeStruct((B, OUT), jnp.float32),
        scratch_shapes=[
            pltpu.VMEM((B, H), jnp.float32),
            pltpu.VMEM((B, 1), jnp.float32),
        ],
    )(parts, denoms, bias, batch3, wc, bc)


def _edge_sweep(h, a, src, dst):
    alpha = a[src, 0] + a[dst, 1]
    alpha = jnp.maximum(alpha, 0.2 * alpha)
    ex = jnp.exp(alpha)
    denom = jax.ops.segment_sum(ex, dst, num_segments=N)
    out = jax.ops.segment_sum(h[src] * ex[:, None], dst, num_segments=N)
    parts = jnp.stack([out, jnp.zeros_like(out)])
    denoms = jnp.broadcast_to(
        jnp.stack([denom, jnp.zeros_like(denom)])[:, :, None], (2, N, 16))
    return parts, denoms


def kernel(x, edge_index, batch, W1, att_src1, att_dst1, b1,
           W2, att_src2, att_dst2, b2, Wc, bc):
    loop = jnp.arange(N, dtype=edge_index.dtype)
    src = jnp.concatenate([edge_index[0], loop])
    dst = jnp.concatenate([edge_index[1], loop])

    att1 = jnp.stack([att_src1, att_dst1], axis=1)
    att2 = jnp.stack([att_src2, att_dst2], axis=1)

    h1, a1 = _tc_front(x, W1, att1)
    parts1, den1 = _edge_sweep(h1, a1, src, dst)
    h2, a2 = _tc_mid(parts1, den1, b1.reshape(1, H), W2, att2)
    parts2, den2 = _edge_sweep(h2, a2, src, dst)
    logits = _tc_final(parts2, den2, b2.reshape(1, H),
                       batch.reshape(MSTEPS, 1, MBLK), Wc,
                       bc.reshape(1, OUT))
    return logits


# speedup vs baseline: 13.0062x; 13.0062x over previous
import dataclasses
import functools

import jax
import jax.numpy as jnp
from jax import lax
from jax.experimental import pallas as pl
from jax.experimental.pallas import tpu as pltpu
from jax.experimental.pallas import tpu_sc as plsc

N = 10000
E = 320000
D = 128
H = 128
B = 64
OUT = 2
E_TOT = E + N

MBLK = 1000
MSTEPS = N // MBLK


def _front_body(x_ref, w_ref, att_ref, hl_ref, hh_ref, a_ref):
    h = jnp.dot(x_ref[...], w_ref[...], preferred_element_type=jnp.float32)
    hl_ref[...] = h[:, :H // 2]
    hh_ref[...] = h[:, H // 2:]
    a_ref[...] = jnp.dot(h, att_ref[...], preferred_element_type=jnp.float32)


_H_OUT_SPECS = [
    pl.BlockSpec((MBLK, H // 2), lambda i: (i, 0)),
    pl.BlockSpec((MBLK, H // 2), lambda i: (i, 0)),
    pl.BlockSpec((MBLK, 2), lambda i: (i, 0)),
]
_H_OUT_SHAPE = [
    jax.ShapeDtypeStruct((N, H // 2), jnp.float32),
    jax.ShapeDtypeStruct((N, H // 2), jnp.float32),
    jax.ShapeDtypeStruct((N, 2), jnp.float32),
]


def _tc_front(x, w, att):
    return pl.pallas_call(
        _front_body,
        grid=(MSTEPS,),
        in_specs=[
            pl.BlockSpec((MBLK, D), lambda i: (i, 0)),
            pl.BlockSpec((D, H), lambda i: (0, 0)),
            pl.BlockSpec((H, 2), lambda i: (0, 0)),
        ],
        out_specs=_H_OUT_SPECS,
        out_shape=_H_OUT_SHAPE,
    )(x, w, att)


def _agg(p_ref, d_ref, b_ref):
    s = jnp.concatenate([p_ref[0], p_ref[1]], axis=-1)
    den = jnp.maximum(d_ref[:, 0:1], 1e-16)
    return jnp.maximum(s / den + b_ref[...], 0.0)


def _mid_body(p_ref, d_ref, b_ref, w_ref, att_ref, hl_ref, hh_ref, a_ref):
    y = _agg(p_ref, d_ref, b_ref)
    h = jnp.dot(y, w_ref[...], preferred_element_type=jnp.float32)
    hl_ref[...] = h[:, :H // 2]
    hh_ref[...] = h[:, H // 2:]
    a_ref[...] = jnp.dot(h, att_ref[...], preferred_element_type=jnp.float32)


def _tc_mid(parts, denom, bias, w, att):
    return pl.pallas_call(
        _mid_body,
        grid=(MSTEPS,),
        in_specs=[
            pl.BlockSpec((2, MBLK, H // 2), lambda i: (0, i, 0)),
            pl.BlockSpec((MBLK, 16), lambda i: (i, 0)),
            pl.BlockSpec((1, H), lambda i: (0, 0)),
            pl.BlockSpec((H, H), lambda i: (0, 0)),
            pl.BlockSpec((H, 2), lambda i: (0, 0)),
        ],
        out_specs=_H_OUT_SPECS,
        out_shape=_H_OUT_SHAPE,
    )(parts, denom, bias, w, att)


def _final_body(p_ref, d_ref, b_ref, batch_ref, wc_ref, bc_ref, o_ref,
                acc_ref, cnt_ref):
    i = pl.program_id(0)

    @pl.when(i == 0)
    def _():
        acc_ref[...] = jnp.zeros_like(acc_ref)
        cnt_ref[...] = jnp.zeros_like(cnt_ref)

    y = _agg(p_ref, d_ref, b_ref)
    seg = batch_ref[0, 0, :]
    oh = (seg[None, :] == lax.broadcasted_iota(jnp.int32, (B, MBLK), 0)
          ).astype(jnp.float32)
    acc_ref[...] += jnp.dot(oh, y, preferred_element_type=jnp.float32)
    cnt_ref[...] += jnp.sum(oh, axis=1, keepdims=True)

    @pl.when(i == MSTEPS - 1)
    def _():
        pooled = acc_ref[...] / jnp.maximum(cnt_ref[...], 1.0)
        o_ref[...] = (jnp.dot(pooled, wc_ref[...],
                              preferred_element_type=jnp.float32)
                      + bc_ref[...])


def _tc_final(parts, denom, bias, batch3, wc, bc):
    return pl.pallas_call(
        _final_body,
        grid=(MSTEPS,),
        in_specs=[
            pl.BlockSpec((2, MBLK, H // 2), lambda i: (0, i, 0)),
            pl.BlockSpec((MBLK, 16), lambda i: (i, 0)),
            pl.BlockSpec((1, H), lambda i: (0, 0)),
            pl.BlockSpec((1, 1, MBLK), lambda i: (i, 0, 0)),
            pl.BlockSpec((H, OUT), lambda i: (0, 0)),
            pl.BlockSpec((1, OUT), lambda i: (0, 0)),
        ],
        out_specs=pl.BlockSpec((B, OUT), lambda i: (0, 0)),
        out_shape=jax.ShapeDtypeStruct((B, OUT), jnp.float32),
        scratch_shapes=[
            pltpu.VMEM((B, H), jnp.float32),
            pltpu.VMEM((B, 1), jnp.float32),
        ],
    )(parts, denom, bias, batch3, wc, bc)


NC, NS = 2, 16
HW = H // 2
K_EDGE = 128
CHUNKS = 162
EPT = K_EDGE * CHUNKS
E_PAD = EPT * NS
ROWS_PT = 632
_G = K_EDGE // 16


def _sc_body(hl_hbm, hh_hbm, as_hbm, ad_hbm, src_hbm, dst_hbm, out_hbm,
             den_hbm, out_sh, den_sh, src_v, dst_v, asv_v, adv_v, rows_v,
             ex_v, sem):
    cid = lax.axis_index("c")
    sid = lax.axis_index("s")
    iota16 = lax.broadcasted_iota(jnp.int32, (16,), 0)
    z16f = jnp.zeros((16,), jnp.float32)

    @pl.loop(0, K_EDGE)
    def _(r):
        ex_v[r, :] = z16f
        for c in range(HW // 16):
            rows_v[r, pl.ds(c * 16, 16)] = z16f

    row0 = jnp.minimum(sid * ROWS_PT, N - ROWS_PT)
    nfull = ROWS_PT // K_EDGE
    rem = ROWS_PT - nfull * K_EDGE
    for t in range(nfull):
        pltpu.sync_copy(rows_v, out_sh.at[pl.ds(row0 + t * K_EDGE, K_EDGE)])
        pltpu.sync_copy(ex_v, den_sh.at[pl.ds(row0 + t * K_EDGE, K_EDGE)])
    pltpu.sync_copy(rows_v.at[pl.ds(0, rem)],
                    out_sh.at[pl.ds(row0 + nfull * K_EDGE, rem)])
    pltpu.sync_copy(ex_v.at[pl.ds(0, rem)],
                    den_sh.at[pl.ds(row0 + nfull * K_EDGE, rem)])

    plsc.subcore_barrier()

    ebase = sid * EPT

    @pl.loop(0, CHUNKS)
    def _(q):
        base = ebase + q * K_EDGE
        pltpu.sync_copy(src_hbm.at[pl.ds(base, K_EDGE)], src_v)
        pltpu.sync_copy(dst_hbm.at[pl.ds(base, K_EDGE)], dst_v)
        pltpu.async_copy(as_hbm.at[src_v], asv_v, sem).wait()
        pltpu.async_copy(ad_hbm.at[dst_v], adv_v, sem).wait()

        @pl.when(cid == 0)
        def _():
            pltpu.async_copy(hl_hbm.at[src_v], rows_v, sem).wait()

        @pl.when(cid != 0)
        def _():
            pltpu.async_copy(hh_hbm.at[src_v], rows_v, sem).wait()

        @pl.loop(0, _G)
        def _(g):
            asv = asv_v[pl.ds(g * 16, 16)]
            adv = adv_v[pl.ds(g * 16, 16)]
            al = asv + adv
            al = jnp.maximum(al, 0.2 * al)
            ex = jnp.exp(al)
            pos = base + g * 16 + iota16
            ex = jnp.where(pos < E_TOT, ex, 0.0)
            for j in range(16):
                spl = z16f + jnp.sum(jnp.where(iota16 == j, ex, 0.0))
                r = g * 16 + j
                ex_v[r, :] = spl
                for c in range(HW // 16):
                    rows_v[r, pl.ds(c * 16, 16)] = (
                        rows_v[r, pl.ds(c * 16, 16)] * spl)

        pltpu.sync_copy(rows_v, out_sh.at[dst_v], add=True)

        @pl.when(cid == 0)
        def _():
            pltpu.sync_copy(ex_v, den_sh.at[dst_v], add=True)

    plsc.subcore_barrier()
    pltpu.sync_copy(out_sh.at[pl.ds(row0, ROWS_PT)],
                    out_hbm.at[cid, pl.ds(row0, ROWS_PT)])

    @pl.when(cid == 0)
    def _():
        pltpu.sync_copy(den_sh.at[pl.ds(row0, ROWS_PT)],
                        den_hbm.at[pl.ds(row0, ROWS_PT)])


def _edge_sweep(hl, hh, a_src, a_dst, src_pad, dst_pad):
    mesh = plsc.VectorSubcoreMesh(core_axis_name="c", subcore_axis_name="s")
    cp = pltpu.CompilerParams(use_tc_tiling_on_sc=False)
    if "needs_layout_passes" in pltpu.CompilerParams.__dataclass_fields__:
        cp = dataclasses.replace(cp, needs_layout_passes=False)
    f = pl.kernel(
        _sc_body,
        out_type=[
            jax.ShapeDtypeStruct((2, N, HW), jnp.float32),
            jax.ShapeDtypeStruct((N, 16), jnp.float32),
        ],
        mesh=mesh,
        scratch_types=[
            pltpu.VMEM_SHARED((N, HW), jnp.float32),
            pltpu.VMEM_SHARED((N, 16), jnp.float32),
            pltpu.VMEM((K_EDGE,), jnp.int32),
            pltpu.VMEM((K_EDGE,), jnp.int32),
            pltpu.VMEM((K_EDGE,), jnp.float32),
            pltpu.VMEM((K_EDGE,), jnp.float32),
            pltpu.VMEM((K_EDGE, HW), jnp.float32),
            pltpu.VMEM((K_EDGE, 16), jnp.float32),
            pltpu.SemaphoreType.DMA,
        ],
        compiler_params=cp,
    )
    return f(hl, hh, a_src, a_dst, src_pad, dst_pad)


def kernel(x, edge_index, batch, W1, att_src1, att_dst1, b1,
           W2, att_src2, att_dst2, b2, Wc, bc):
    loop = jnp.arange(N, dtype=edge_index.dtype)
    pad = jnp.arange(E_PAD - E_TOT, dtype=edge_index.dtype) % N
    src = jnp.concatenate([edge_index[0], loop, pad])
    dst = jnp.concatenate([edge_index[1], loop, pad])

    att1 = jnp.stack([att_src1, att_dst1], axis=1)
    att2 = jnp.stack([att_src2, att_dst2], axis=1)

    hl1, hh1, a1 = _tc_front(x, W1, att1)
    parts1, den1 = _edge_sweep(hl1, hh1, a1[:, 0], a1[:, 1], src, dst)
    hl2, hh2, a2 = _tc_mid(parts1, den1, b1.reshape(1, H), W2, att2)
    parts2, den2 = _edge_sweep(hl2, hh2, a2[:, 0], a2[:, 1], src, dst)
    logits = _tc_final(parts2, den2, b2.reshape(1, H),
                       batch.reshape(MSTEPS, 1, MBLK), Wc,
                       bc.reshape(1, OUT))
    return logits


# speedup vs baseline: 25.7997x; 1.9836x over previous
import dataclasses
import functools

import jax
import jax.numpy as jnp
from jax import lax
from jax.experimental import pallas as pl
from jax.experimental.pallas import tpu as pltpu
from jax.experimental.pallas import tpu_sc as plsc

N = 10000
E = 320000
D = 128
H = 128
B = 64
OUT = 2
E_TOT = E + N

MBLK = 1000
MSTEPS = N // MBLK


def _front_body(x_ref, w_ref, att_ref, hl_ref, hh_ref, a_ref):
    h = jnp.dot(x_ref[...], w_ref[...], preferred_element_type=jnp.float32)
    hl_ref[...] = h[:, :H // 2]
    hh_ref[...] = h[:, H // 2:]
    a_ref[...] = jnp.dot(h, att_ref[...], preferred_element_type=jnp.float32)


_H_OUT_SPECS = [
    pl.BlockSpec((MBLK, H // 2), lambda i: (i, 0)),
    pl.BlockSpec((MBLK, H // 2), lambda i: (i, 0)),
    pl.BlockSpec((MBLK, 2), lambda i: (i, 0)),
]
_H_OUT_SHAPE = [
    jax.ShapeDtypeStruct((N, H // 2), jnp.float32),
    jax.ShapeDtypeStruct((N, H // 2), jnp.float32),
    jax.ShapeDtypeStruct((N, 2), jnp.float32),
]


def _tc_front(x, w, att):
    return pl.pallas_call(
        _front_body,
        grid=(MSTEPS,),
        in_specs=[
            pl.BlockSpec((MBLK, D), lambda i: (i, 0)),
            pl.BlockSpec((D, H), lambda i: (0, 0)),
            pl.BlockSpec((H, 2), lambda i: (0, 0)),
        ],
        out_specs=_H_OUT_SPECS,
        out_shape=_H_OUT_SHAPE,
    )(x, w, att)


def _agg(p_ref, d_ref, b_ref):
    s = jnp.concatenate([p_ref[0], p_ref[1]], axis=-1)
    den = jnp.maximum(d_ref[:, 0:1], 1e-16)
    return jnp.maximum(s / den + b_ref[...], 0.0)


def _mid_body(p_ref, d_ref, b_ref, w_ref, att_ref, hl_ref, hh_ref, a_ref):
    y = _agg(p_ref, d_ref, b_ref)
    h = jnp.dot(y, w_ref[...], preferred_element_type=jnp.float32)
    hl_ref[...] = h[:, :H // 2]
    hh_ref[...] = h[:, H // 2:]
    a_ref[...] = jnp.dot(h, att_ref[...], preferred_element_type=jnp.float32)


def _tc_mid(parts, denom, bias, w, att):
    return pl.pallas_call(
        _mid_body,
        grid=(MSTEPS,),
        in_specs=[
            pl.BlockSpec((2, MBLK, H // 2), lambda i: (0, i, 0)),
            pl.BlockSpec((MBLK, 16), lambda i: (i, 0)),
            pl.BlockSpec((1, H), lambda i: (0, 0)),
            pl.BlockSpec((H, H), lambda i: (0, 0)),
            pl.BlockSpec((H, 2), lambda i: (0, 0)),
        ],
        out_specs=_H_OUT_SPECS,
        out_shape=_H_OUT_SHAPE,
    )(parts, denom, bias, w, att)


def _final_body(p_ref, d_ref, b_ref, batch_ref, wc_ref, bc_ref, o_ref,
                acc_ref, cnt_ref):
    i = pl.program_id(0)

    @pl.when(i == 0)
    def _():
        acc_ref[...] = jnp.zeros_like(acc_ref)
        cnt_ref[...] = jnp.zeros_like(cnt_ref)

    y = _agg(p_ref, d_ref, b_ref)
    seg = batch_ref[0, 0, :]
    oh = (seg[None, :] == lax.broadcasted_iota(jnp.int32, (B, MBLK), 0)
          ).astype(jnp.float32)
    acc_ref[...] += jnp.dot(oh, y, preferred_element_type=jnp.float32)
    cnt_ref[...] += jnp.sum(oh, axis=1, keepdims=True)

    @pl.when(i == MSTEPS - 1)
    def _():
        pooled = acc_ref[...] / jnp.maximum(cnt_ref[...], 1.0)
        o_ref[...] = (jnp.dot(pooled, wc_ref[...],
                              preferred_element_type=jnp.float32)
                      + bc_ref[...])


def _tc_final(parts, denom, bias, batch3, wc, bc):
    return pl.pallas_call(
        _final_body,
        grid=(MSTEPS,),
        in_specs=[
            pl.BlockSpec((2, MBLK, H // 2), lambda i: (0, i, 0)),
            pl.BlockSpec((MBLK, 16), lambda i: (i, 0)),
            pl.BlockSpec((1, H), lambda i: (0, 0)),
            pl.BlockSpec((1, 1, MBLK), lambda i: (i, 0, 0)),
            pl.BlockSpec((H, OUT), lambda i: (0, 0)),
            pl.BlockSpec((1, OUT), lambda i: (0, 0)),
        ],
        out_specs=pl.BlockSpec((B, OUT), lambda i: (0, 0)),
        out_shape=jax.ShapeDtypeStruct((B, OUT), jnp.float32),
        scratch_shapes=[
            pltpu.VMEM((B, H), jnp.float32),
            pltpu.VMEM((B, 1), jnp.float32),
        ],
    )(parts, denom, bias, batch3, wc, bc)


NC, NS = 2, 16
HW = H // 2
K_EDGE = 80
CHUNKS = 258
EPT = K_EDGE * CHUNKS
E_PAD = EPT * NS
ROWS_PT = 632
_G = K_EDGE // 16


def _sc_body(hl_hbm, hh_hbm, att2_hbm, idx3_hbm, out_hbm, den_hbm,
             out_sh, den_sh, slot0, slot1, sidx0, sidx1, av0, av1,
             rows0, rows1, ex0, ex1, isem0, isem1, gsem0, gsem1,
             ssem0, ssem1):
    cid = lax.axis_index("c")
    sid = lax.axis_index("s")
    iota16 = lax.broadcasted_iota(jnp.int32, (16,), 0)
    z16f = jnp.zeros((16,), jnp.float32)
    slots = (slot0, slot1)
    sidxs = (sidx0, sidx1)
    avs = (av0, av1)
    rows = (rows0, rows1)
    exs = (ex0, ex1)
    isems = (isem0, isem1)
    gsems = (gsem0, gsem1)
    ssems = (ssem0, ssem1)

    @pl.loop(0, K_EDGE)
    def _(r):
        ex0[r, :] = z16f
        for c in range(HW // 16):
            rows0[r, pl.ds(c * 16, 16)] = z16f

    row0 = jnp.minimum(sid * ROWS_PT, N - ROWS_PT)
    nfull = ROWS_PT // K_EDGE
    rem = ROWS_PT - nfull * K_EDGE
    for t in range(nfull):
        pltpu.sync_copy(rows0, out_sh.at[pl.ds(row0 + t * K_EDGE, K_EDGE)])
        pltpu.sync_copy(ex0, den_sh.at[pl.ds(row0 + t * K_EDGE, K_EDGE)])
    pltpu.sync_copy(rows0.at[pl.ds(0, rem)],
                    out_sh.at[pl.ds(row0 + nfull * K_EDGE, rem)])
    pltpu.sync_copy(ex0.at[pl.ds(0, rem)],
                    den_sh.at[pl.ds(row0 + nfull * K_EDGE, rem)])

    plsc.subcore_barrier()

    ebase = sid * EPT

    def issue_idx(q, s):
        pltpu.async_copy(idx3_hbm.at[sid, q], slots[s], isems[s])

    def wait_idx(s):
        pltpu.make_async_copy(idx3_hbm.at[sid, 0], slots[s], isems[s]).wait()

    def issue_gathers(b):
        sl = slots[b]
        pltpu.async_copy(att2_hbm.at[sl.at[1]], avs[b].at[0], gsems[b])
        pltpu.async_copy(att2_hbm.at[sl.at[2]], avs[b].at[1], gsems[b])

        @pl.when(cid == 0)
        def _():
            pltpu.async_copy(hl_hbm.at[sl.at[1]], rows[b], gsems[b])

        @pl.when(cid != 0)
        def _():
            pltpu.async_copy(hh_hbm.at[sl.at[1]], rows[b], gsems[b])

    def wait_gathers(b):
        sl = slots[b]
        pltpu.make_async_copy(att2_hbm.at[sl.at[1]], avs[b].at[0],
                              gsems[b]).wait()
        pltpu.make_async_copy(att2_hbm.at[sl.at[2]], avs[b].at[1],
                              gsems[b]).wait()

        @pl.when(cid == 0)
        def _():
            pltpu.make_async_copy(hl_hbm.at[sl.at[1]], rows[b],
                                  gsems[b]).wait()

        @pl.when(cid != 0)
        def _():
            pltpu.make_async_copy(hh_hbm.at[sl.at[1]], rows[b],
                                  gsems[b]).wait()

    def copy_sidx(b):
        for g in range(_G):
            sidxs[b][0, pl.ds(g * 16, 16)] = slots[b][0, pl.ds(g * 16, 16)]

    def compute(b, base):
        @pl.loop(0, _G)
        def _(g):
            asv = avs[b][0, pl.ds(g * 16, 16)]
            adv = avs[b][1, pl.ds(g * 16, 16)]
            al = asv + adv
            al = jnp.maximum(al, 0.2 * al)
            ex = jnp.exp(al)
            pos = base + g * 16 + iota16
            ex = jnp.where(pos < E_TOT, ex, 0.0)
            for j in range(16):
                spl = z16f + jnp.sum(jnp.where(iota16 == j, ex, 0.0))
                r = g * 16 + j
                exs[b][r, :] = spl
                for c in range(HW // 16):
                    rows[b][r, pl.ds(c * 16, 16)] = (
                        rows[b][r, pl.ds(c * 16, 16)] * spl)

    def issue_scatters(b):
        pltpu.async_copy(rows[b], out_sh.at[sidxs[b].at[0]], ssems[b],
                         add=True)

        @pl.when(cid == 0)
        def _():
            pltpu.async_copy(exs[b], den_sh.at[sidxs[b].at[0]], ssems[b],
                             add=True)

    def wait_scatters(b):
        pltpu.make_async_copy(rows[b], out_sh.at[sidxs[b].at[0]],
                              ssems[b]).wait()

        @pl.when(cid == 0)
        def _():
            pltpu.make_async_copy(exs[b], den_sh.at[sidxs[b].at[0]],
                                  ssems[b]).wait()

    issue_idx(0, 0)
    issue_idx(1, 1)
    wait_idx(0)
    issue_gathers(0)

    HALF = CHUNKS // 2

    @pl.loop(0, HALF)
    def _(qq):
        q0 = qq * 2

        @pl.when(qq >= 1)
        def _():
            wait_scatters(1)

        wait_idx(1)
        issue_gathers(1)

        @pl.when(qq < HALF - 1)
        def _():
            issue_idx(q0 + 2, 0)

        wait_gathers(0)
        copy_sidx(0)
        compute(0, ebase + q0 * K_EDGE)
        issue_scatters(0)

        q1 = q0 + 1

        @pl.when(qq < HALF - 1)
        def _():
            wait_scatters(0)
            wait_idx(0)
            issue_gathers(0)
            issue_idx(q1 + 2, 1)

        wait_gathers(1)
        copy_sidx(1)
        compute(1, ebase + q1 * K_EDGE)
        issue_scatters(1)

    wait_scatters(0)
    wait_scatters(1)

    plsc.subcore_barrier()
    pltpu.sync_copy(out_sh.at[pl.ds(row0, ROWS_PT)],
                    out_hbm.at[cid, pl.ds(row0, ROWS_PT)])

    @pl.when(cid == 0)
    def _():
        pltpu.sync_copy(den_sh.at[pl.ds(row0, ROWS_PT)],
                        den_hbm.at[pl.ds(row0, ROWS_PT)])


def _edge_sweep(hl, hh, att2, idx3):
    mesh = plsc.VectorSubcoreMesh(core_axis_name="c", subcore_axis_name="s")
    cp = pltpu.CompilerParams(use_tc_tiling_on_sc=False)
    if "needs_layout_passes" in pltpu.CompilerParams.__dataclass_fields__:
        cp = dataclasses.replace(cp, needs_layout_passes=False)
    f = pl.kernel(
        _sc_body,
        out_type=[
            jax.ShapeDtypeStruct((2, N, HW), jnp.float32),
            jax.ShapeDtypeStruct((N, 16), jnp.float32),
        ],
        mesh=mesh,
        scratch_types=[
            pltpu.VMEM_SHARED((N, HW), jnp.float32),
            pltpu.VMEM_SHARED((N, 16), jnp.float32),
            pltpu.VMEM((3, K_EDGE), jnp.int32),
            pltpu.VMEM((3, K_EDGE), jnp.int32),
            pltpu.VMEM((1, K_EDGE), jnp.int32),
            pltpu.VMEM((1, K_EDGE), jnp.int32),
            pltpu.VMEM((2, K_EDGE), jnp.float32),
            pltpu.VMEM((2, K_EDGE), jnp.float32),
            pltpu.VMEM((K_EDGE, HW), jnp.float32),
            pltpu.VMEM((K_EDGE, HW), jnp.float32),
            pltpu.VMEM((K_EDGE, 16), jnp.float32),
            pltpu.VMEM((K_EDGE, 16), jnp.float32),
            pltpu.SemaphoreType.DMA,
            pltpu.SemaphoreType.DMA,
            pltpu.SemaphoreType.DMA,
            pltpu.SemaphoreType.DMA,
            pltpu.SemaphoreType.DMA,
            pltpu.SemaphoreType.DMA,
        ],
        compiler_params=cp,
    )
    return f(hl, hh, att2, idx3)


def kernel(x, edge_index, batch, W1, att_src1, att_dst1, b1,
           W2, att_src2, att_dst2, b2, Wc, bc):
    loop = jnp.arange(N, dtype=edge_index.dtype)
    pad = jnp.arange(E_PAD - E_TOT, dtype=edge_index.dtype) % N
    src = jnp.concatenate([edge_index[0], loop, pad])
    dst = jnp.concatenate([edge_index[1], loop, pad])
    src_r = src.reshape(NS, CHUNKS, K_EDGE)
    dst_r = dst.reshape(NS, CHUNKS, K_EDGE)
    idx3 = jnp.stack([dst_r, src_r, dst_r + N], axis=2)

    att1 = jnp.stack([att_src1, att_dst1], axis=1)
    att2 = jnp.stack([att_src2, att_dst2], axis=1)

    hl1, hh1, a1 = _tc_front(x, W1, att1)
    att2_1 = jnp.concatenate([a1[:, 0], a1[:, 1]])
    parts1, den1 = _edge_sweep(hl1, hh1, att2_1, idx3)
    hl2, hh2, a2 = _tc_mid(parts1, den1, b1.reshape(1, H), W2, att2)
    att2_2 = jnp.concatenate([a2[:, 0], a2[:, 1]])
    parts2, den2 = _edge_sweep(hl2, hh2, att2_2, idx3)
    logits = _tc_final(parts2, den2, b2.reshape(1, H),
                       batch.reshape(MSTEPS, 1, MBLK), Wc,
                       bc.reshape(1, OUT))
    return logits


# speedup vs baseline: 41.3107x; 1.6012x over previous
import dataclasses
import functools

import jax
import jax.numpy as jnp
from jax import lax
from jax.experimental import pallas as pl
from jax.experimental.pallas import tpu as pltpu
from jax.experimental.pallas import tpu_sc as plsc

N = 10000
E = 320000
D = 128
H = 128
B = 64
OUT = 2
E_TOT = E + N

MBLK = 1000
MSTEPS = N // MBLK


def _front_body(x_ref, w_ref, att_ref, hl_ref, hh_ref, a_ref):
    h = jnp.dot(x_ref[...], w_ref[...], preferred_element_type=jnp.float32,
                 precision=lax.Precision.HIGHEST)
    hl_ref[...] = h[:, :H // 2]
    hh_ref[...] = h[:, H // 2:]
    a_ref[...] = jnp.dot(h, att_ref[...], preferred_element_type=jnp.float32,
                 precision=lax.Precision.HIGHEST)


_H_OUT_SPECS = [
    pl.BlockSpec((MBLK, H // 2), lambda i: (i, 0)),
    pl.BlockSpec((MBLK, H // 2), lambda i: (i, 0)),
    pl.BlockSpec((MBLK, 2), lambda i: (i, 0)),
]
_H_OUT_SHAPE = [
    jax.ShapeDtypeStruct((N, H // 2), jnp.float32),
    jax.ShapeDtypeStruct((N, H // 2), jnp.float32),
    jax.ShapeDtypeStruct((N, 2), jnp.float32),
]


def _tc_front(x, w, att):
    return pl.pallas_call(
        _front_body,
        grid=(MSTEPS,),
        in_specs=[
            pl.BlockSpec((MBLK, D), lambda i: (i, 0)),
            pl.BlockSpec((D, H), lambda i: (0, 0)),
            pl.BlockSpec((H, 2), lambda i: (0, 0)),
        ],
        out_specs=_H_OUT_SPECS,
        out_shape=_H_OUT_SHAPE,
    )(x, w, att)


def _agg(p_ref, d_ref, b_ref):
    s = jnp.concatenate([p_ref[0], p_ref[1]], axis=-1)
    den = jnp.maximum(d_ref[:, 0:1], 1e-16)
    return jnp.maximum(s / den + b_ref[...], 0.0)


def _mid_body(p_ref, d_ref, b_ref, w_ref, att_ref, hl_ref, hh_ref, a_ref):
    y = _agg(p_ref, d_ref, b_ref)
    h = jnp.dot(y, w_ref[...], preferred_element_type=jnp.float32,
                 precision=lax.Precision.HIGHEST)
    hl_ref[...] = h[:, :H // 2]
    hh_ref[...] = h[:, H // 2:]
    a_ref[...] = jnp.dot(h, att_ref[...], preferred_element_type=jnp.float32,
                 precision=lax.Precision.HIGHEST)


def _tc_mid(parts, denom, bias, w, att):
    return pl.pallas_call(
        _mid_body,
        grid=(MSTEPS,),
        in_specs=[
            pl.BlockSpec((2, MBLK, H // 2), lambda i: (0, i, 0)),
            pl.BlockSpec((MBLK, 16), lambda i: (i, 0)),
            pl.BlockSpec((1, H), lambda i: (0, 0)),
            pl.BlockSpec((H, H), lambda i: (0, 0)),
            pl.BlockSpec((H, 2), lambda i: (0, 0)),
        ],
        out_specs=_H_OUT_SPECS,
        out_shape=_H_OUT_SHAPE,
    )(parts, denom, bias, w, att)


def _final_body(p_ref, d_ref, b_ref, batch_ref, wc_ref, bc_ref, o_ref,
                acc_ref, cnt_ref):
    i = pl.program_id(0)

    @pl.when(i == 0)
    def _():
        acc_ref[...] = jnp.zeros_like(acc_ref)
        cnt_ref[...] = jnp.zeros_like(cnt_ref)

    y = _agg(p_ref, d_ref, b_ref)
    seg = batch_ref[0, 0, :]
    oh = (seg[None, :] == lax.broadcasted_iota(jnp.int32, (B, MBLK), 0)
          ).astype(jnp.float32)
    acc_ref[...] += jnp.dot(oh, y, preferred_element_type=jnp.float32,
                 precision=lax.Precision.HIGHEST)
    cnt_ref[...] += jnp.sum(oh, axis=1, keepdims=True)

    @pl.when(i == MSTEPS - 1)
    def _():
        pooled = acc_ref[...] / jnp.maximum(cnt_ref[...], 1.0)
        o_ref[...] = (jnp.dot(pooled, wc_ref[...],
                              preferred_element_type=jnp.float32,
                 precision=lax.Precision.HIGHEST)
                      + bc_ref[...])


def _tc_final(parts, denom, bias, batch3, wc, bc):
    return pl.pallas_call(
        _final_body,
        grid=(MSTEPS,),
        in_specs=[
            pl.BlockSpec((2, MBLK, H // 2), lambda i: (0, i, 0)),
            pl.BlockSpec((MBLK, 16), lambda i: (i, 0)),
            pl.BlockSpec((1, H), lambda i: (0, 0)),
            pl.BlockSpec((1, 1, MBLK), lambda i: (i, 0, 0)),
            pl.BlockSpec((H, OUT), lambda i: (0, 0)),
            pl.BlockSpec((1, OUT), lambda i: (0, 0)),
        ],
        out_specs=pl.BlockSpec((B, OUT), lambda i: (0, 0)),
        out_shape=jax.ShapeDtypeStruct((B, OUT), jnp.float32),
        scratch_shapes=[
            pltpu.VMEM((B, H), jnp.float32),
            pltpu.VMEM((B, 1), jnp.float32),
        ],
    )(parts, denom, bias, batch3, wc, bc)


NC, NS = 2, 16
HW = H // 2
K_EDGE = 80
CHUNKS = 258
EPT = K_EDGE * CHUNKS
E_PAD = EPT * NS
ROWS_PT = 632
_G = K_EDGE // 16


def _sc_body(hl_hbm, hh_hbm, att2_hbm, idx3_hbm, out_hbm, den_hbm,
             out_sh, den_sh, slot0, slot1, sidx0, sidx1, av0, av1,
             rows0, rows1, ex0, ex1, isem0, isem1, gsem0, gsem1,
             ssem0, ssem1):
    cid = lax.axis_index("c")
    sid = lax.axis_index("s")
    iota16 = lax.broadcasted_iota(jnp.int32, (16,), 0)
    z16f = jnp.zeros((16,), jnp.float32)
    slots = (slot0, slot1)
    sidxs = (sidx0, sidx1)
    avs = (av0, av1)
    rows = (rows0, rows1)
    exs = (ex0, ex1)
    isems = (isem0, isem1)
    gsems = (gsem0, gsem1)
    ssems = (ssem0, ssem1)

    @pl.loop(0, K_EDGE)
    def _(r):
        ex0[r, :] = z16f
        for c in range(HW // 16):
            rows0[r, pl.ds(c * 16, 16)] = z16f

    row0 = jnp.minimum(sid * ROWS_PT, N - ROWS_PT)
    nfull = ROWS_PT // K_EDGE
    rem = ROWS_PT - nfull * K_EDGE
    for t in range(nfull):
        pltpu.sync_copy(rows0, out_sh.at[pl.ds(row0 + t * K_EDGE, K_EDGE)])
        pltpu.sync_copy(ex0, den_sh.at[pl.ds(row0 + t * K_EDGE, K_EDGE)])
    pltpu.sync_copy(rows0.at[pl.ds(0, rem)],
                    out_sh.at[pl.ds(row0 + nfull * K_EDGE, rem)])
    pltpu.sync_copy(ex0.at[pl.ds(0, rem)],
                    den_sh.at[pl.ds(row0 + nfull * K_EDGE, rem)])

    plsc.subcore_barrier()

    ebase = sid * EPT

    def issue_idx(q, s):
        pltpu.async_copy(idx3_hbm.at[sid, q], slots[s], isems[s])

    def wait_idx(s):
        pltpu.make_async_copy(idx3_hbm.at[sid, 0], slots[s], isems[s]).wait()

    def issue_gathers(b):
        sl = slots[b]
        pltpu.async_copy(att2_hbm.at[sl.at[1]], avs[b].at[0], gsems[b])
        pltpu.async_copy(att2_hbm.at[sl.at[2]], avs[b].at[1], gsems[b])

        @pl.when(cid == 0)
        def _():
            pltpu.async_copy(hl_hbm.at[sl.at[1]], rows[b], gsems[b])

        @pl.when(cid != 0)
        def _():
            pltpu.async_copy(hh_hbm.at[sl.at[1]], rows[b], gsems[b])

    def wait_gathers(b):
        sl = slots[b]
        pltpu.make_async_copy(att2_hbm.at[sl.at[1]], avs[b].at[0],
                              gsems[b]).wait()
        pltpu.make_async_copy(att2_hbm.at[sl.at[2]], avs[b].at[1],
                              gsems[b]).wait()

        @pl.when(cid == 0)
        def _():
            pltpu.make_async_copy(hl_hbm.at[sl.at[1]], rows[b],
                                  gsems[b]).wait()

        @pl.when(cid != 0)
        def _():
            pltpu.make_async_copy(hh_hbm.at[sl.at[1]], rows[b],
                                  gsems[b]).wait()

    def copy_sidx(b):
        for g in range(_G):
            sidxs[b][0, pl.ds(g * 16, 16)] = slots[b][0, pl.ds(g * 16, 16)]

    def compute(b, base):
        @pl.loop(0, _G)
        def _(g):
            asv = avs[b][0, pl.ds(g * 16, 16)]
            adv = avs[b][1, pl.ds(g * 16, 16)]
            al = asv + adv
            al = jnp.maximum(al, 0.2 * al)
            ex = jnp.exp(al)
            pos = base + g * 16 + iota16
            ex = jnp.where(pos < E_TOT, ex, 0.0)
            spls = [z16f + jnp.sum(jnp.where(iota16 == j, ex, 0.0))
                    for j in range(16)]
            for j in range(16):
                exs[b][g * 16 + j, :] = spls[j]
            for c in range(HW // 16):
                for j in range(16):
                    r = g * 16 + j
                    rows[b][r, pl.ds(c * 16, 16)] = (
                        rows[b][r, pl.ds(c * 16, 16)] * spls[j])

    def issue_scatters(b):
        pltpu.async_copy(rows[b], out_sh.at[sidxs[b].at[0]], ssems[b],
                         add=True)

        @pl.when(cid == 0)
        def _():
            pltpu.async_copy(exs[b], den_sh.at[sidxs[b].at[0]], ssems[b],
                             add=True)

    def wait_scatters(b):
        pltpu.make_async_copy(rows[b], out_sh.at[sidxs[b].at[0]],
                              ssems[b]).wait()

        @pl.when(cid == 0)
        def _():
            pltpu.make_async_copy(exs[b], den_sh.at[sidxs[b].at[0]],
                                  ssems[b]).wait()

    issue_idx(0, 0)
    issue_idx(1, 1)
    wait_idx(0)
    issue_gathers(0)

    HALF = CHUNKS // 2

    @pl.loop(0, HALF)
    def _(qq):
        q0 = qq * 2

        @pl.when(qq >= 1)
        def _():
            wait_scatters(1)

        wait_idx(1)
        issue_gathers(1)
        wait_gathers(0)
        copy_sidx(0)

        @pl.when(qq < HALF - 1)
        def _():
            issue_idx(q0 + 2, 0)

        compute(0, ebase + q0 * K_EDGE)
        issue_scatters(0)

        q1 = q0 + 1

        @pl.when(qq < HALF - 1)
        def _():
            wait_scatters(0)
            wait_idx(0)
            issue_gathers(0)

        wait_gathers(1)
        copy_sidx(1)

        @pl.when(qq < HALF - 1)
        def _():
            issue_idx(q1 + 2, 1)

        compute(1, ebase + q1 * K_EDGE)
        issue_scatters(1)

    wait_scatters(0)
    wait_scatters(1)

    plsc.subcore_barrier()
    pltpu.sync_copy(out_sh.at[pl.ds(row0, ROWS_PT)],
                    out_hbm.at[cid, pl.ds(row0, ROWS_PT)])

    @pl.when(cid == 0)
    def _():
        pltpu.sync_copy(den_sh.at[pl.ds(row0, ROWS_PT)],
                        den_hbm.at[pl.ds(row0, ROWS_PT)])


def _edge_sweep(hl, hh, att2, idx3):
    mesh = plsc.VectorSubcoreMesh(core_axis_name="c", subcore_axis_name="s")
    cp = pltpu.CompilerParams(use_tc_tiling_on_sc=False)
    if "needs_layout_passes" in pltpu.CompilerParams.__dataclass_fields__:
        cp = dataclasses.replace(cp, needs_layout_passes=False)
    f = pl.kernel(
        _sc_body,
        out_type=[
            jax.ShapeDtypeStruct((2, N, HW), jnp.float32),
            jax.ShapeDtypeStruct((N, 16), jnp.float32),
        ],
        mesh=mesh,
        scratch_types=[
            pltpu.VMEM_SHARED((N, HW), jnp.float32),
            pltpu.VMEM_SHARED((N, 16), jnp.float32),
            pltpu.VMEM((3, K_EDGE), jnp.int32),
            pltpu.VMEM((3, K_EDGE), jnp.int32),
            pltpu.VMEM((1, K_EDGE), jnp.int32),
            pltpu.VMEM((1, K_EDGE), jnp.int32),
            pltpu.VMEM((2, K_EDGE), jnp.float32),
            pltpu.VMEM((2, K_EDGE), jnp.float32),
            pltpu.VMEM((K_EDGE, HW), jnp.float32),
            pltpu.VMEM((K_EDGE, HW), jnp.float32),
            pltpu.VMEM((K_EDGE, 16), jnp.float32),
            pltpu.VMEM((K_EDGE, 16), jnp.float32),
            pltpu.SemaphoreType.DMA,
            pltpu.SemaphoreType.DMA,
            pltpu.SemaphoreType.DMA,
            pltpu.SemaphoreType.DMA,
            pltpu.SemaphoreType.DMA,
            pltpu.SemaphoreType.DMA,
        ],
        compiler_params=cp,
    )
    return f(hl, hh, att2, idx3)


def kernel(x, edge_index, batch, W1, att_src1, att_dst1, b1,
           W2, att_src2, att_dst2, b2, Wc, bc):
    loop = jnp.arange(N, dtype=edge_index.dtype)
    pad = jnp.arange(E_PAD - E_TOT, dtype=edge_index.dtype) % N
    src = jnp.concatenate([edge_index[0], loop, pad])
    dst = jnp.concatenate([edge_index[1], loop, pad])
    src_r = src.reshape(NS, CHUNKS, K_EDGE)
    dst_r = dst.reshape(NS, CHUNKS, K_EDGE)
    idx3 = jnp.stack([dst_r, src_r, dst_r + N], axis=2)

    att1 = jnp.stack([att_src1, att_dst1], axis=1)
    att2 = jnp.stack([att_src2, att_dst2], axis=1)

    hl1, hh1, a1 = _tc_front(x, W1, att1)
    att2_1 = jnp.concatenate([a1[:, 0], a1[:, 1]])
    parts1, den1 = _edge_sweep(hl1, hh1, att2_1, idx3)
    hl2, hh2, a2 = _tc_mid(parts1, den1, b1.reshape(1, H), W2, att2)
    att2_2 = jnp.concatenate([a2[:, 0], a2[:, 1]])
    parts2, den2 = _edge_sweep(hl2, hh2, att2_2, idx3)
    logits = _tc_final(parts2, den2, b2.reshape(1, H),
                       batch.reshape(MSTEPS, 1, MBLK), Wc,
                       bc.reshape(1, OUT))
    return logits
